# Initial kernel scaffold; baseline (speedup 1.0000x reference)
import functools
import numpy as np
import jax
import jax.numpy as jnp
from jax.experimental import pallas as pl
from jax.experimental.pallas import tpu as pltpu


def _fold_bn(gamma, beta, mean, var, eps=1e-5):
    scale = gamma / jnp.sqrt(var + eps)
    shift = beta - mean * scale
    return scale, shift


def _block_diag(w_oihw, groups):
    cout, cin_g = w_oihw.shape[0], w_oihw.shape[1]
    oc_g = cout // groups
    W = jnp.zeros((cin_g * groups, cout), jnp.float32)
    for g in range(groups):
        blk = w_oihw[g * oc_g:(g + 1) * oc_g, :, 0, 0]
        W = W.at[g * cin_g:(g + 1) * cin_g, g * oc_g:(g + 1) * oc_g].set(blk.T)
    return W


def _unit_kernel(x_ref, w1t_ref, sc1_ref, sh1_ref, wdw_ref, sc2_ref, sh2_ref,
                 w2t_ref, sc3_ref, sh3_ref, o_ref, buf_ref, *, h, w, ksize):
    hw = h * w
    pad = ksize // 2
    P = 32

    x = x_ref[...]
    y = jnp.dot(w1t_ref[...], x.astype(jnp.bfloat16),
                preferred_element_type=jnp.float32)
    y = jnp.maximum(y * sc1_ref[...] + sh1_ref[...], 0.0)

    buf_ref[...] = jnp.zeros_like(buf_ref)
    buf_ref[:, P:P + hw] = y
    col = jax.lax.broadcasted_iota(jnp.int32, (1, hw), 1) % w

    acc = None
    for kw in range(ksize):
        part = None
        for kh in range(ksize):
            d = (kh - pad) * w + (kw - pad)
            k = kh * ksize + kw
            term = buf_ref[:, P + d:P + d + hw] * wdw_ref[:, k:k + 1]
            part = term if part is None else part + term
        if kw < pad:
            part = part * (col >= (pad - kw)).astype(jnp.float32)
        elif kw > pad:
            part = part * (col < w - (kw - pad)).astype(jnp.float32)
        acc = part if acc is None else acc + part

    z = acc * sc2_ref[...] + sh2_ref[...]

    out = jnp.dot(w2t_ref[...], z.astype(jnp.bfloat16),
                  preferred_element_type=jnp.float32)
    out = jnp.maximum(out * sc3_ref[...] + sh3_ref[...] + x, 0.0)
    o_ref[...] = out


def kernel(x, w1, wdw, w2,
           bn1_gamma, bn1_beta, bn1_mean, bn1_var,
           bn2_gamma, bn2_beta, bn2_mean, bn2_var,
           bn3_gamma, bn3_beta, bn3_mean, bn3_var):
    inp, oup, group = 256, 256, 4
    mid, ksize = 256, 3
    n, cin, h, w = x.shape
    assert cin == inp and oup == inp
    hw = h * w

    W1 = _block_diag(w1, group)
    W2 = _block_diag(w2, group)
    gc = mid // group
    perm = np.arange(mid).reshape(gc, group).T.reshape(-1)
    W2 = W2[np.argsort(perm), :]
    w1t = W1.T.astype(jnp.bfloat16)
    w2t = W2.T.astype(jnp.bfloat16)

    sc1, sh1 = _fold_bn(bn1_gamma, bn1_beta, bn1_mean, bn1_var)
    sc2, sh2 = _fold_bn(bn2_gamma, bn2_beta, bn2_mean, bn2_var)
    sc3, sh3 = _fold_bn(bn3_gamma, bn3_beta, bn3_mean, bn3_var)
    wdw_cm = wdw[:, 0, :, :].reshape(mid, ksize * ksize)

    xcm = x.reshape(n, inp, hw)

    kern = functools.partial(_unit_kernel, h=h, w=w, ksize=ksize)
    out = pl.pallas_call(
        kern,
        out_shape=jax.ShapeDtypeStruct((n, oup, hw), jnp.float32),
        grid=(n,),
        in_specs=[
            pl.BlockSpec((None, inp, hw), lambda i: (i, 0, 0)),
            pl.BlockSpec((mid, inp), lambda i: (0, 0)),
            pl.BlockSpec((mid, 1), lambda i: (0, 0)),
            pl.BlockSpec((mid, 1), lambda i: (0, 0)),
            pl.BlockSpec((mid, ksize * ksize), lambda i: (0, 0)),
            pl.BlockSpec((mid, 1), lambda i: (0, 0)),
            pl.BlockSpec((mid, 1), lambda i: (0, 0)),
            pl.BlockSpec((oup, mid), lambda i: (0, 0)),
            pl.BlockSpec((oup, 1), lambda i: (0, 0)),
            pl.BlockSpec((oup, 1), lambda i: (0, 0)),
        ],
        out_specs=pl.BlockSpec((None, oup, hw), lambda i: (i, 0, 0)),
        scratch_shapes=[pltpu.VMEM((mid, hw + 64), jnp.float32)],
        compiler_params=pltpu.CompilerParams(
            dimension_semantics=("parallel",),
            vmem_limit_bytes=int(32 << 20)),
    )(xcm, w1t, sc1[:, None], sh1[:, None], wdw_cm, sc2[:, None],
      sh2[:, None], w2t, sc3[:, None], sh3[:, None])
    return out.reshape(n, oup, h, w)


# speedup vs baseline: 1.0812x; 1.0812x over previous
import functools
import numpy as np
import jax
import jax.numpy as jnp
from jax.experimental import pallas as pl
from jax.experimental.pallas import tpu as pltpu


def _fold_bn(gamma, beta, mean, var, eps=1e-5):
    scale = gamma / jnp.sqrt(var + eps)
    shift = beta - mean * scale
    return scale, shift


def _block_diag(w_oihw, groups):
    cout, cin_g = w_oihw.shape[0], w_oihw.shape[1]
    oc_g = cout // groups
    W = jnp.zeros((cin_g * groups, cout), jnp.float32)
    for g in range(groups):
        blk = w_oihw[g * oc_g:(g + 1) * oc_g, :, 0, 0]
        W = W.at[g * cin_g:(g + 1) * cin_g, g * oc_g:(g + 1) * oc_g].set(blk.T)
    return W


def _unit_kernel(x_ref, w1t_ref, b1_ref, wdw_ref, w2t_ref, b3_ref,
                 o_ref, buf_ref, bufa_ref, bufb_ref, *, h, w, ksize):
    hw = h * w
    P = 128

    x = x_ref[...]
    y = jnp.dot(w1t_ref[...], x.astype(jnp.bfloat16),
                preferred_element_type=jnp.float32)
    y = jnp.maximum(y + b1_ref[...], 0.0)

    zpad = jnp.zeros((y.shape[0], 32), jnp.float32)
    buf_ref[:, P - 32:P] = zpad
    buf_ref[:, P + hw:P + hw + 32] = zpad
    buf_ref[:, P:P + hw] = y
    col = jax.lax.broadcasted_iota(jnp.int32, (1, hw), 1) % w

    t0 = buf_ref[:, P - 1:P - 1 + hw] * (col >= 1).astype(jnp.float32)
    t1 = y
    t2 = buf_ref[:, P + 1:P + 1 + hw] * (col <= w - 2).astype(jnp.float32)

    def urow(kh):
        return (t0 * wdw_ref[:, 3 * kh:3 * kh + 1]
                + t1 * wdw_ref[:, 3 * kh + 1:3 * kh + 2]
                + t2 * wdw_ref[:, 3 * kh + 2:3 * kh + 3])

    bufa_ref[:, P - 32:P] = zpad
    bufa_ref[:, P:P + hw] = urow(0)
    bufb_ref[:, P + hw:P + hw + 32] = zpad
    bufb_ref[:, P:P + hw] = urow(2)
    z = (urow(1) + bufa_ref[:, P - w:P - w + hw]
         + bufb_ref[:, P + w:P + w + hw])

    out = jnp.dot(w2t_ref[...], z.astype(jnp.bfloat16),
                  preferred_element_type=jnp.float32)
    out = jnp.maximum(out + b3_ref[...] + x, 0.0)
    o_ref[...] = out


def kernel(x, w1, wdw, w2,
           bn1_gamma, bn1_beta, bn1_mean, bn1_var,
           bn2_gamma, bn2_beta, bn2_mean, bn2_var,
           bn3_gamma, bn3_beta, bn3_mean, bn3_var):
    inp, oup, group = 256, 256, 4
    mid, ksize = 256, 3
    n, cin, h, w = x.shape
    assert cin == inp and oup == inp
    hw = h * w

    W1 = _block_diag(w1, group)
    W2 = _block_diag(w2, group)
    gc = mid // group
    perm = np.arange(mid).reshape(gc, group).T.reshape(-1)
    W2 = W2[np.argsort(perm), :]

    sc1, sh1 = _fold_bn(bn1_gamma, bn1_beta, bn1_mean, bn1_var)
    sc2, sh2 = _fold_bn(bn2_gamma, bn2_beta, bn2_mean, bn2_var)
    sc3, sh3 = _fold_bn(bn3_gamma, bn3_beta, bn3_mean, bn3_var)

    w1t = (W1.T * sc1[:, None]).astype(jnp.bfloat16)
    w2t = (W2.T * sc3[:, None]).astype(jnp.bfloat16)
    b1 = sh1[:, None]
    b3 = (sc3 * (W2.T @ sh2) + sh3)[:, None]
    wdw_cm = (wdw[:, 0, :, :].reshape(mid, ksize * ksize)
              * sc2[:, None])

    xcm = x.reshape(n, inp, hw)

    kern = functools.partial(_unit_kernel, h=h, w=w, ksize=ksize)
    out = pl.pallas_call(
        kern,
        out_shape=jax.ShapeDtypeStruct((n, oup, hw), jnp.float32),
        grid=(n,),
        in_specs=[
            pl.BlockSpec((None, inp, hw), lambda i: (i, 0, 0)),
            pl.BlockSpec((mid, inp), lambda i: (0, 0)),
            pl.BlockSpec((mid, 1), lambda i: (0, 0)),
            pl.BlockSpec((mid, ksize * ksize), lambda i: (0, 0)),
            pl.BlockSpec((oup, mid), lambda i: (0, 0)),
            pl.BlockSpec((oup, 1), lambda i: (0, 0)),
        ],
        out_specs=pl.BlockSpec((None, oup, hw), lambda i: (i, 0, 0)),
        scratch_shapes=[pltpu.VMEM((mid, hw + 160), jnp.float32),
                        pltpu.VMEM((mid, hw + 160), jnp.float32),
                        pltpu.VMEM((mid, hw + 160), jnp.float32)],
        compiler_params=pltpu.CompilerParams(
            dimension_semantics=("parallel",),
            vmem_limit_bytes=int(32 << 20)),
    )(xcm, w1t, b1, wdw_cm, w2t, b3)
    return out.reshape(n, oup, h, w)


# speedup vs baseline: 1.7622x; 1.6299x over previous
import functools
import numpy as np
import jax
import jax.numpy as jnp
from jax.experimental import pallas as pl
from jax.experimental.pallas import tpu as pltpu


def _fold_bn(gamma, beta, mean, var, eps=1e-5):
    scale = gamma / jnp.sqrt(var + eps)
    shift = beta - mean * scale
    return scale, shift


def _block_diag(w_oihw, groups):
    cout, cin_g = w_oihw.shape[0], w_oihw.shape[1]
    oc_g = cout // groups
    W = jnp.zeros((cin_g * groups, cout), jnp.float32)
    for g in range(groups):
        blk = w_oihw[g * oc_g:(g + 1) * oc_g, :, 0, 0]
        W = W.at[g * cin_g:(g + 1) * cin_g, g * oc_g:(g + 1) * oc_g].set(blk.T)
    return W


def _unit_kernel(x_ref, w1t_ref, b1_ref, wdw_ref, w2t_ref, b3_ref,
                 o_ref, *, h, w, ksize, nsub):
    hw = h * w
    col = jax.lax.broadcasted_iota(jnp.int32, (1, hw), 1) % w
    maskl = (col >= 1).astype(jnp.bfloat16)
    maskr = (col <= w - 2).astype(jnp.bfloat16)

    for j in range(nsub):
        x = x_ref[j]
        y = jnp.dot(w1t_ref[...], x.astype(jnp.bfloat16),
                    preferred_element_type=jnp.float32)
        y = jnp.maximum(y + b1_ref[...], 0.0).astype(jnp.bfloat16)

        c = y.shape[0]
        z1 = jnp.zeros((c, 1), jnp.bfloat16)
        zw = jnp.zeros((c, w), jnp.bfloat16)
        t0 = jnp.concatenate([z1, y[:, :hw - 1]], 1) * maskl
        t1 = y
        t2 = jnp.concatenate([y[:, 1:], z1], 1) * maskr

        def urow(kh):
            return (t0 * wdw_ref[:, 3 * kh:3 * kh + 1]
                    + t1 * wdw_ref[:, 3 * kh + 1:3 * kh + 2]
                    + t2 * wdw_ref[:, 3 * kh + 2:3 * kh + 3])

        u0, u2 = urow(0), urow(2)
        z = (urow(1) + jnp.concatenate([zw, u0[:, :hw - w]], 1)
             + jnp.concatenate([u2[:, w:], zw], 1))

        out = jnp.dot(w2t_ref[...], z,
                      preferred_element_type=jnp.float32)
        out = jnp.maximum(out + b3_ref[...] + x, 0.0)
        o_ref[j] = out


def kernel(x, w1, wdw, w2,
           bn1_gamma, bn1_beta, bn1_mean, bn1_var,
           bn2_gamma, bn2_beta, bn2_mean, bn2_var,
           bn3_gamma, bn3_beta, bn3_mean, bn3_var):
    inp, oup, group = 256, 256, 4
    mid, ksize = 256, 3
    n, cin, h, w = x.shape
    assert cin == inp and oup == inp
    hw = h * w

    W1 = _block_diag(w1, group)
    W2 = _block_diag(w2, group)
    gc = mid // group
    perm = np.arange(mid).reshape(gc, group).T.reshape(-1)
    W2 = W2[np.argsort(perm), :]

    sc1, sh1 = _fold_bn(bn1_gamma, bn1_beta, bn1_mean, bn1_var)
    sc2, sh2 = _fold_bn(bn2_gamma, bn2_beta, bn2_mean, bn2_var)
    sc3, sh3 = _fold_bn(bn3_gamma, bn3_beta, bn3_mean, bn3_var)

    w1t = (W1.T * sc1[:, None]).astype(jnp.bfloat16)
    w2t = (W2.T * sc3[:, None]).astype(jnp.bfloat16)
    b1 = sh1[:, None]
    b3 = (sc3 * (W2.T @ sh2) + sh3)[:, None]
    wdw_cm = (wdw[:, 0, :, :].reshape(mid, ksize * ksize)
              * sc2[:, None]).astype(jnp.bfloat16)

    xcm = x.reshape(n, inp, hw)

    nsub = 1
    kern = functools.partial(_unit_kernel, h=h, w=w, ksize=ksize, nsub=nsub)
    out = pl.pallas_call(
        kern,
        out_shape=jax.ShapeDtypeStruct((n, oup, hw), jnp.float32),
        grid=(n // nsub,),
        in_specs=[
            pl.BlockSpec((nsub, inp, hw), lambda i: (i, 0, 0)),
            pl.BlockSpec((mid, inp), lambda i: (0, 0)),
            pl.BlockSpec((mid, 1), lambda i: (0, 0)),
            pl.BlockSpec((mid, ksize * ksize), lambda i: (0, 0)),
            pl.BlockSpec((oup, mid), lambda i: (0, 0)),
            pl.BlockSpec((oup, 1), lambda i: (0, 0)),
        ],
        out_specs=pl.BlockSpec((nsub, oup, hw), lambda i: (i, 0, 0)),
        compiler_params=pltpu.CompilerParams(
            dimension_semantics=("parallel",),
            vmem_limit_bytes=int(32 << 20)),
    )(xcm, w1t, b1, wdw_cm, w2t, b3)
    return out.reshape(n, oup, h, w)
